# baseline (speedup 1.0000x reference)
import functools

import jax
import jax.numpy as jnp
from jax.experimental import pallas as pl

K_NEIGHBORS = 32
Q = 1024
N_REFS = 100000
D = 64


def _dist_tile_kernel(q_ref, r_ref, rsq_ref, out_ref):
    q = q_ref[...]
    r = r_ref[...]
    qsq = jnp.sum(q * q, axis=1, keepdims=True)
    dot = jax.lax.dot_general(
        q, r, (((1,), (1,)), ((), ())),
        preferred_element_type=jnp.float32,
    )
    out_ref[...] = qsq + rsq_ref[...] - 2.0 * dot


def kernel(queries, refs):
    k = K_NEIGHBORS
    n = refs.shape[0]
    N_PAD = 100352
    BR = 2048
    n_blocks = N_PAD // BR
    refs_p = jnp.pad(refs, ((0, N_PAD - n), (0, 0)))
    rsq = jnp.sum(refs_p * refs_p, axis=1)
    rsq = jnp.where(jnp.arange(N_PAD) < n, rsq, jnp.float32(3e38))[None, :]

    sq_dist = pl.pallas_call(
        _dist_tile_kernel,
        grid=(n_blocks,),
        in_specs=[
            pl.BlockSpec((Q, D), lambda i: (0, 0)),
            pl.BlockSpec((BR, D), lambda i: (i, 0)),
            pl.BlockSpec((1, BR), lambda i: (0, i)),
        ],
        out_specs=pl.BlockSpec((Q, BR), lambda i: (0, i)),
        out_shape=jax.ShapeDtypeStruct((Q, N_PAD), jnp.float32),
    )(queries, refs_p, rsq)

    neg_top, knn_index = jax.lax.top_k(-sq_dist, k)
    knn_dist = jnp.sqrt(jnp.maximum(-neg_top, 0.0))
    return knn_dist, knn_index


# speedup vs baseline: 7.2150x; 7.2150x over previous
import functools

import jax
import jax.numpy as jnp
from jax import lax
from jax.experimental import pallas as pl
from jax.experimental.pallas import tpu as pltpu
from jax.experimental.pallas import tpu_sc as plsc

K_NEIGHBORS = 32
Q = 1024
D_DIM = 64
N_PAD = 100352
BR = 2048
NBLK = N_PAD // BR
NGROUP = N_PAD // 128
GPB = BR // 128
PAD_VAL = 1e30

NTILES = 32
QPT = Q // NTILES
NCHUNK_M = NGROUP // 16
NCHUNK_G = (32 * 128) // 16


def _dist_tile_kernel(q_ref, rsq_ref, r_ref, out_ref, m_ref):
    q = q_ref[...]
    r = r_ref[...]
    qsq = jnp.sum(q * q, axis=1, keepdims=True)
    dot = lax.dot_general(q, r, (((1,), (1,)), ((), ())),
                          preferred_element_type=jnp.float32)
    d = qsq + rsq_ref[...] - 2.0 * dot
    out_ref[...] = d
    m = jnp.min(d.reshape(Q, GPB, 128), axis=-1)
    m_ref[...] = m[None]


def _merge16(A, Ai, B, Bi, v, vi, tmp16):
    C, Ci = plsc.sort_key_val(v, vi)
    rC = lax.rev(C, (0,))
    rCi = lax.rev(Ci, (0,))
    sel = B <= rC
    L1 = jnp.minimum(B, rC)
    L1i = jnp.where(sel, Bi, rCi)
    L1, L1i = plsc.sort_key_val(L1, L1i)
    rL = lax.rev(L1, (0,))
    rLi = lax.rev(L1i, (0,))
    sel2 = A <= rL
    lo = jnp.minimum(A, rL)
    loi = jnp.where(sel2, Ai, rLi)
    hi = jnp.maximum(A, rL)
    hii = jnp.where(sel2, rLi, Ai)
    A2, A2i = plsc.sort_key_val(lo, loi)
    B2, B2i = plsc.sort_key_val(hi, hii)
    tmp16[...] = B2
    bmaxv = plsc.load_gather(tmp16, [jnp.full((16,), 15, jnp.int32)])
    return A2, A2i, B2, B2i, bmaxv


def _sc_select(m_hbm, dflat_hbm, outv_hbm, outi_hbm,
               mrows, gat, idxv, gbase, tmp16, outv, outi, sem):
    wid = lax.axis_index("s") * 2 + lax.axis_index("c")
    qbase = wid * QPT
    pltpu.sync_copy(m_hbm.at[pl.ds(qbase, QPT)], mrows)
    iota = lax.iota(jnp.int32, 16)
    big = jnp.full((16,), PAD_VAL, jnp.float32)
    zero = jnp.zeros((16,), jnp.int32)

    def per_query(qi, _):
        def stepB(j, car):
            A, Ai, B, Bi, bmaxv = car
            v = mrows[qi, pl.ds(j * 16, 16)]

            def do(car):
                A, Ai, B, Bi, _ = car
                return _merge16(A, Ai, B, Bi, v, j * 16 + iota, tmp16)

            return lax.cond(jnp.any(v < bmaxv), do, lambda c: c, car)

        carB = (big, zero, big, zero, big)
        A, Ai, B, Bi, bmaxv = lax.fori_loop(0, NCHUNK_M, stepB, carB)

        row0 = (qbase + qi) * NGROUP
        idxv[pl.ds(0, 16)] = row0 + Ai
        idxv[pl.ds(16, 16)] = row0 + Bi
        gbase[pl.ds(0, 16)] = Ai * 128
        gbase[pl.ds(16, 16)] = Bi * 128
        pltpu.async_copy(dflat_hbm.at[idxv], gat, sem).wait()

        def stepD(c, car):
            A, Ai, B, Bi, bmaxv = car
            s = c // 8
            o = (c % 8) * 16
            v = gat[s, pl.ds(o, 16)]

            def do(car):
                A, Ai, B, Bi, _ = car
                base = plsc.load_gather(gbase, [jnp.full((16,), s, jnp.int32)])
                return _merge16(A, Ai, B, Bi, v, base + o + iota, tmp16)

            return lax.cond(jnp.any(v < bmaxv), do, lambda c2: c2, car)

        carD = (big, zero, big, zero, big)
        A, Ai, B, Bi, bmaxv = lax.fori_loop(0, NCHUNK_G, stepD, carD)

        outv[qi, pl.ds(0, 16)] = A
        outv[qi, pl.ds(16, 16)] = B
        outi[qi, pl.ds(0, 16)] = Ai
        outi[qi, pl.ds(16, 16)] = Bi
        return 0

    lax.fori_loop(0, QPT, per_query, 0)
    pltpu.sync_copy(outv, outv_hbm.at[pl.ds(qbase, QPT)])
    pltpu.sync_copy(outi, outi_hbm.at[pl.ds(qbase, QPT)])


def kernel(queries, refs):
    n = refs.shape[0]
    refs_p = jnp.pad(refs, ((0, N_PAD - n), (0, 0)))
    rsq = jnp.sum(refs_p * refs_p, axis=1)
    rsq = jnp.where(jnp.arange(N_PAD) < n, rsq, jnp.float32(PAD_VAL))[None, :]

    sq_dist, m3 = pl.pallas_call(
        _dist_tile_kernel,
        grid=(NBLK,),
        in_specs=[
            pl.BlockSpec((Q, D_DIM), lambda i: (0, 0)),
            pl.BlockSpec((1, BR), lambda i: (0, i)),
            pl.BlockSpec((BR, D_DIM), lambda i: (i, 0)),
        ],
        out_specs=[
            pl.BlockSpec((Q, BR), lambda i: (0, i)),
            pl.BlockSpec((1, Q, GPB), lambda i: (i, 0, 0)),
        ],
        out_shape=[
            jax.ShapeDtypeStruct((Q, N_PAD), jnp.float32),
            jax.ShapeDtypeStruct((NBLK, Q, GPB), jnp.float32),
        ],
    )(queries, rsq, refs_p)

    m2 = m3.transpose(1, 0, 2).reshape(Q, NGROUP)
    dflat = sq_dist.reshape(Q * NGROUP, 128)

    mesh = plsc.VectorSubcoreMesh(core_axis_name="c", subcore_axis_name="s")
    sc = pl.kernel(
        _sc_select,
        mesh=mesh,
        compiler_params=pltpu.CompilerParams(needs_layout_passes=False),
        out_type=[
            jax.ShapeDtypeStruct((Q, K_NEIGHBORS), jnp.float32),
            jax.ShapeDtypeStruct((Q, K_NEIGHBORS), jnp.int32),
        ],
        scratch_types=[
            pltpu.VMEM((QPT, NGROUP), jnp.float32),
            pltpu.VMEM((32, 128), jnp.float32),
            pltpu.VMEM((32,), jnp.int32),
            pltpu.VMEM((32,), jnp.int32),
            pltpu.VMEM((16,), jnp.float32),
            pltpu.VMEM((QPT, K_NEIGHBORS), jnp.float32),
            pltpu.VMEM((QPT, K_NEIGHBORS), jnp.int32),
            pltpu.SemaphoreType.DMA,
        ],
    )
    sqd, idx = sc(m2, dflat)
    return jnp.sqrt(jnp.maximum(sqd, 0.0)), idx
